# baseline (speedup 1.0000x reference)
import functools

import jax
import jax.numpy as jnp
from jax import lax
from jax.experimental import pallas as pl
from jax.experimental.pallas import tpu as pltpu
from jax.experimental.pallas import tpu_sc as plsc

EMB = 64
NUM_CORES = 2
NUM_SUBCORES = 16
NW = NUM_CORES * NUM_SUBCORES

CHUNK = 128
NBUF = 8


@functools.partial(jax.jit, static_argnames=())
def _embedding_lookup(flat_ids, table):
    n = flat_ids.shape[0]
    n_per_w = n // NW
    n_chunks = n_per_w // CHUNK
    idx2d = flat_ids.reshape(NW * n_chunks, CHUNK)

    mesh = plsc.VectorSubcoreMesh(
        core_axis_name="c", subcore_axis_name="s",
        num_cores=NUM_CORES, num_subcores=NUM_SUBCORES)

    @functools.partial(
        pl.kernel,
        mesh=mesh,
        out_type=jax.ShapeDtypeStruct((n, EMB), jnp.float32),
        scratch_types=[
            pltpu.VMEM((n_chunks, CHUNK), jnp.int32),
            pltpu.VMEM((NBUF, CHUNK, EMB), jnp.float32),
            pltpu.SemaphoreType.DMA((NBUF,)),
        ],
        compiler_params=pltpu.CompilerParams(use_tc_tiling_on_sc=False),
    )
    def emb_kernel(ids_hbm, table_hbm, out_hbm, idx_v, rows_v, gsem):
        wid = lax.axis_index("s") * NUM_CORES + lax.axis_index("c")
        base_chunk = wid * n_chunks
        pltpu.sync_copy(ids_hbm.at[pl.ds(base_chunk, n_chunks)], idx_v)

        def start_gather(g, b):
            pltpu.async_copy(table_hbm.at[idx_v.at[g]], rows_v.at[b], gsem.at[b])

        def finish_chunk(g, b):
            pltpu.make_async_copy(table_hbm.at[idx_v.at[g]], rows_v.at[b],
                                  gsem.at[b]).wait()
            off = (base_chunk + g) * CHUNK
            pltpu.sync_copy(rows_v.at[b], out_hbm.at[pl.ds(off, CHUNK)])

        for b in range(NBUF):
            start_gather(b, b)

        @pl.loop(0, n_chunks - NBUF, step=NBUF)
        def _main(j):
            for b in range(NBUF):
                g = j + b
                finish_chunk(g, b)
                start_gather(g + NBUF, b)

        for b in range(NBUF):
            finish_chunk(n_chunks - NBUF + b, b)

    return emb_kernel(idx2d, table)


def kernel(input_ids, table):
    b, s = input_ids.shape
    flat_ids = input_ids.reshape(b * s).astype(jnp.int32)
    out = _embedding_lookup(flat_ids, table)
    return out.reshape(b, s, EMB)


# speedup vs baseline: 1.0009x; 1.0009x over previous
import functools

import jax
import jax.numpy as jnp
from jax import lax
from jax.experimental import pallas as pl
from jax.experimental.pallas import tpu as pltpu
from jax.experimental.pallas import tpu_sc as plsc

EMB = 64
NUM_CORES = 2
NUM_SUBCORES = 16
NW = NUM_CORES * NUM_SUBCORES

CHUNK = 128
NBUF = 8


@functools.partial(jax.jit, static_argnames=())
def _embedding_lookup(flat_ids, table):
    n = flat_ids.shape[0]
    n_per_w = n // NW
    n_chunks = n_per_w // CHUNK
    idx2d = flat_ids.reshape(NW * n_chunks, CHUNK)

    mesh = plsc.VectorSubcoreMesh(
        core_axis_name="c", subcore_axis_name="s",
        num_cores=NUM_CORES, num_subcores=NUM_SUBCORES)

    @functools.partial(
        pl.kernel,
        mesh=mesh,
        out_type=jax.ShapeDtypeStruct((n, EMB), jnp.float32),
        scratch_types=[
            pltpu.VMEM((n_chunks, CHUNK), jnp.int32),
            pltpu.VMEM((NBUF, CHUNK, EMB), jnp.float32),
            pltpu.SemaphoreType.DMA((NBUF,)),
            pltpu.SemaphoreType.DMA((NBUF,)),
        ],
        compiler_params=pltpu.CompilerParams(use_tc_tiling_on_sc=False),
    )
    def emb_kernel(ids_hbm, table_hbm, out_hbm, idx_v, rows_v, gsem, osem):
        wid = lax.axis_index("s") * NUM_CORES + lax.axis_index("c")
        base_chunk = wid * n_chunks
        pltpu.sync_copy(ids_hbm.at[pl.ds(base_chunk, n_chunks)], idx_v)

        def start_gather(g, b):
            pltpu.async_copy(table_hbm.at[idx_v.at[g]], rows_v.at[b], gsem.at[b])

        def wait_gather(g, b):
            pltpu.make_async_copy(table_hbm.at[idx_v.at[g]], rows_v.at[b],
                                  gsem.at[b]).wait()

        def start_out(g, b):
            off = (base_chunk + g) * CHUNK
            pltpu.async_copy(rows_v.at[b], out_hbm.at[pl.ds(off, CHUNK)],
                             osem.at[b])

        def wait_out(b):
            pltpu.make_async_copy(rows_v.at[b], out_hbm.at[pl.ds(0, CHUNK)],
                                  osem.at[b]).wait()

        D = NBUF // 2

        for g in range(D):
            start_gather(g, g % NBUF)
        for h in range(D):
            start_gather(h + D, (h + D) % NBUF)
            wait_gather(h, h % NBUF)
            start_out(h, h % NBUF)

        @pl.loop(D, n_chunks - D, step=NBUF)
        def _main(j):
            for i in range(NBUF):
                h = j + i
                bp = (h + D) % NBUF
                wait_out(bp)
                start_gather(h + D, bp)
                wait_gather(h, h % NBUF)
                start_out(h, h % NBUF)

        for g in range(n_chunks - D, n_chunks):
            wait_gather(g, g % NBUF)
            start_out(g, g % NBUF)
        for b in range(NBUF):
            wait_out(b)

    return emb_kernel(idx2d, table)


def kernel(input_ids, table):
    b, s = input_ids.shape
    flat_ids = input_ids.reshape(b * s).astype(jnp.int32)
    out = _embedding_lookup(flat_ids, table)
    return out.reshape(b, s, EMB)


# speedup vs baseline: 1.2197x; 1.2186x over previous
import functools

import jax
import jax.numpy as jnp
from jax import lax
from jax.experimental import pallas as pl
from jax.experimental.pallas import tpu as pltpu
from jax.experimental.pallas import tpu_sc as plsc

EMB = 64
PAD_W = 128
NUM_CORES = 2
NUM_SUBCORES = 16
NW = NUM_CORES * NUM_SUBCORES

CHUNK = 128
NBUF = 4


def _embedding_lookup(idx2d, table_p):
    n_chunks_total, _ = idx2d.shape
    n = n_chunks_total * CHUNK
    n_chunks = n_chunks_total // NW

    mesh = plsc.VectorSubcoreMesh(
        core_axis_name="c", subcore_axis_name="s",
        num_cores=NUM_CORES, num_subcores=NUM_SUBCORES)

    @functools.partial(
        pl.kernel,
        mesh=mesh,
        out_type=jax.ShapeDtypeStruct((n, PAD_W), jnp.float32),
        scratch_types=[
            pltpu.VMEM((n_chunks, CHUNK), jnp.int32),
            pltpu.VMEM((NBUF, CHUNK, PAD_W), jnp.float32),
            pltpu.SemaphoreType.DMA((NBUF,)),
            pltpu.SemaphoreType.DMA((NBUF,)),
        ],
    )
    def emb_kernel(ids_hbm, table_hbm, out_hbm, idx_v, rows_v, gsem, osem):
        wid = lax.axis_index("s") * NUM_CORES + lax.axis_index("c")
        base_chunk = wid * n_chunks
        pltpu.sync_copy(ids_hbm.at[pl.ds(base_chunk, n_chunks)], idx_v)

        def start_gather(g, b):
            pltpu.async_copy(table_hbm.at[idx_v.at[g]], rows_v.at[b], gsem.at[b])

        def wait_gather(g, b):
            pltpu.make_async_copy(table_hbm.at[idx_v.at[g]], rows_v.at[b],
                                  gsem.at[b]).wait()

        def start_out(g, b):
            off = (base_chunk + g) * CHUNK
            pltpu.async_copy(rows_v.at[b],
                             out_hbm.at[pl.ds(off, CHUNK)], osem.at[b])

        def wait_out(b):
            pltpu.make_async_copy(rows_v.at[b],
                                  out_hbm.at[pl.ds(0, CHUNK)],
                                  osem.at[b]).wait()

        D = NBUF // 2

        for g in range(D):
            start_gather(g, g % NBUF)
        for h in range(D):
            start_gather(h + D, (h + D) % NBUF)
            wait_gather(h, h % NBUF)
            start_out(h, h % NBUF)

        @pl.loop(D, n_chunks - D, step=NBUF)
        def _main(j):
            for i in range(NBUF):
                h = j + i
                bp = (h + D) % NBUF
                wait_out(bp)
                start_gather(h + D, bp)
                wait_gather(h, h % NBUF)
                start_out(h, h % NBUF)

        for g in range(n_chunks - D, n_chunks):
            wait_gather(g, g % NBUF)
            start_out(g, g % NBUF)
        for b in range(NBUF):
            wait_out(b)

    return emb_kernel(idx2d, table_p)


def kernel(input_ids, table):
    b, s = input_ids.shape
    n = b * s
    idx2d = input_ids.astype(jnp.int32).reshape(n // CHUNK, CHUNK)
    table_p = jnp.pad(table, ((0, 0), (0, PAD_W - EMB)))
    out = _embedding_lookup(idx2d, table_p)
    return out.reshape(b, s, PAD_W)[:, :, :EMB]
